# baseline (speedup 1.0000x reference)
import functools

import jax
import jax.numpy as jnp
from jax import lax
from jax.experimental import pallas as pl
from jax.experimental.pallas import tpu as pltpu
from jax.experimental.pallas import tpu_sc as plsc

N = 10000
NDEN = 10240
E = 320000
D = 128
NC = 2
NS = 16
NW = NC * NS
SUB = 64
NSUB = 10
CHUNK = SUB * NSUB
NSTRIPE = E // CHUNK
ITERS = (NSTRIPE + NW - 1) // NW
NGRP = SUB // 16
ROWS_PER_TILE = N // NS
NV = D // 16


def _mm_body(h_ref, wT_ref, a2_ref, z_ref, s2_ref):
    z = lax.dot_general(
        h_ref[...], wT_ref[...], (((1,), (0,)), ((), ())),
        precision=lax.Precision.HIGHEST, preferred_element_type=jnp.float32)
    s2_ref[...] = lax.dot_general(
        z, a2_ref[...], (((1,), (0,)), ((), ())),
        precision=lax.Precision.HIGHEST, preferred_element_type=jnp.float32)
    z_ref[...] = z


def _matmuls(h, W_fcT, A2):
    return pl.pallas_call(
        _mm_body,
        out_shape=[
            jax.ShapeDtypeStruct((N, D), jnp.float32),
            jax.ShapeDtypeStruct((N, 2), jnp.float32),
        ],
    )(h, W_fcT, A2)


def _sc_body(z_hbm, ssrc_hbm, sdst_hbm, src_hbm, dst_hbm, emb_hbm, cvec_hbm,
             u_out, denx_out, u_acc, den_acc, ssrc_v, sdst_v,
             srcq, dstq, embb, wb, cvb, denl, rowsA, rowsB, rowsFA, rowsFB,
             isem, gsemA, gsemB, ssemA, ssemB, dsem):
    cc = lax.axis_index("c")
    tid = lax.axis_index("s")
    wid = cc * NS + tid

    pltpu.sync_copy(ssrc_hbm, ssrc_v)
    pltpu.sync_copy(sdst_hbm, sdst_v)
    pltpu.sync_copy(cvec_hbm, cvb)
    cval = cvb[...]

    def _zrow(r, _):
        for k in range(NV):
            rowsFA[r, pl.ds(k * 16, 16)] = jnp.zeros((16,), jnp.float32)
        return _
    lax.fori_loop(0, SUB, _zrow, None)
    r0 = tid * ROWS_PER_TILE
    for q in range(ROWS_PER_TILE // SUB):
        pltpu.sync_copy(rowsFA, u_acc.at[pl.ds(r0 + q * SUB, SUB)])
    rem = ROWS_PER_TILE % SUB
    pltpu.sync_copy(rowsFA.at[pl.ds(0, rem)],
                    u_acc.at[pl.ds(r0 + (ROWS_PER_TILE // SUB) * SUB, rem)])
    d0 = tid * (NDEN // NS)
    for q in range(NDEN // NS // D):
        pltpu.sync_copy(rowsFA.at[0], den_acc.at[pl.ds(d0 + q * D, D)])
    plsc.subcore_barrier()

    rbufs = [rowsA, rowsB]
    fbufs = [rowsFA, rowsFB]
    gsems = [gsemA, gsemB]
    ssems = [ssemA, ssemB]
    sqs = [srcq.at[q] for q in range(NSUB)]
    dqs = [dstq.at[q] for q in range(NSUB)]

    def _chunk(jj, _):
        j = jj * NW + wid

        @pl.when(j < NSTRIPE)
        def _():
            off = j * CHUNK
            hs = [pltpu.async_copy(src_hbm.at[pl.ds(off + q * SUB, SUB)],
                                   sqs[q], isem) for q in range(NSUB)]
            hs += [pltpu.async_copy(dst_hbm.at[pl.ds(off + q * SUB, SUB)],
                                    dqs[q], isem) for q in range(NSUB)]
            hs.append(pltpu.async_copy(emb_hbm.at[pl.ds(off, CHUNK)], embb,
                                       isem))
            for h in hs:
                h.wait()

            gh = [None] * NSUB
            sh = [None] * NSUB
            dh = [None] * NSUB
            gh[0] = pltpu.async_copy(z_hbm.at[sqs[0]], rbufs[0], gsems[0])
            for q in range(NSUB):
                rb = rbufs[q % 2]
                fb = fbufs[q % 2]
                if q + 1 < NSUB:
                    gh[q + 1] = pltpu.async_copy(
                        z_hbm.at[sqs[q + 1]], rbufs[(q + 1) % 2],
                        gsems[(q + 1) % 2])

                for t in range(NGRP):
                    t16 = t * 16
                    sv = sqs[q][pl.ds(t16, 16)]
                    dv = dqs[q][pl.ds(t16, 16)]
                    s1 = plsc.load_gather(ssrc_v, [sv])
                    s2 = plsc.load_gather(sdst_v, [dv])
                    em = embb[pl.ds(q * SUB + t16, 16)]
                    e = s1 + s2 + cval * em
                    e = jnp.where(e >= 0.0, e, e * jnp.float32(0.01))
                    wb[pl.ds(q * SUB + t16, 16)] = jnp.exp(e)

                dh[q] = pltpu.async_copy(wb.at[pl.ds(q * SUB, SUB)],
                                         den_acc.at[dqs[q]], dsem, add=True)
                gh[q].wait()
                if q >= 2:
                    sh[q - 2].wait()

                def _scale(r2, _s):
                    r = r2 * 2
                    for rr in range(2):
                        wr = plsc.load_gather(
                            wb,
                            [jnp.zeros((16,), jnp.int32) + (q * SUB + r + rr)])
                        for k in range(4):
                            x = rb[r + rr, pl.ds(k * 32, 32)]
                            xi = plsc.bitcast(x, jnp.int32)
                            lo = plsc.bitcast(
                                lax.shift_left(xi, 16), jnp.float32)
                            hi = plsc.bitcast(
                                jnp.bitwise_and(xi, jnp.int32(-65536)),
                                jnp.float32)
                            fb[r + rr, pl.ds(k * 32, 16)] = lo * wr
                            fb[r + rr, pl.ds(k * 32 + 16, 16)] = hi * wr
                    return _s
                lax.fori_loop(0, SUB // 2, _scale, None)

                sh[q] = pltpu.async_copy(fb, u_acc.at[dqs[q]],
                                         ssems[q % 2], add=True)
            sh[NSUB - 2].wait()
            sh[NSUB - 1].wait()
            for q in range(NSUB):
                dh[q].wait()
        return _

    lax.fori_loop(0, ITERS, _chunk, None)
    plsc.subcore_barrier()

    pltpu.sync_copy(u_acc.at[pl.ds(r0, ROWS_PER_TILE)],
                    u_out.at[cc, pl.ds(r0, ROWS_PER_TILE)])

    a0 = (r0 // 8) * 8
    doff = r0 - a0
    pltpu.sync_copy(den_acc.at[pl.ds(a0, 632)], denl)
    nblk = ROWS_PER_TILE // SUB
    for b in range(nblk + 1):
        cnt = SUB if b < nblk else ROWS_PER_TILE % SUB
        def _exp(r, _, b=b, cnt=cnt):
            dv = plsc.load_gather(
                denl, [jnp.zeros((16,), jnp.int32) + (doff + b * SUB + r)])
            for k in range(NV):
                rowsFA[r, pl.ds(k * 16, 16)] = dv
            return _
        lax.fori_loop(0, cnt, _exp, None)
        pltpu.sync_copy(rowsFA.at[pl.ds(0, cnt)],
                        denx_out.at[cc, pl.ds(r0 + b * SUB, cnt)])


_sc_edges = functools.partial(
    pl.kernel,
    out_type=[
        jax.ShapeDtypeStruct((NC, N, D), jnp.float32),
        jax.ShapeDtypeStruct((NC, N, D), jnp.float32),
    ],
    mesh=plsc.VectorSubcoreMesh(core_axis_name="c", subcore_axis_name="s"),
    compiler_params=pltpu.CompilerParams(
        use_tc_tiling_on_sc=False, needs_layout_passes=False),
    scratch_types=[
        pltpu.VMEM_SHARED((N, D), jnp.float32),
        pltpu.VMEM_SHARED((NDEN,), jnp.float32),
        pltpu.VMEM((N,), jnp.float32),
        pltpu.VMEM((N,), jnp.float32),
        pltpu.VMEM((NSUB, SUB), jnp.int32),
        pltpu.VMEM((NSUB, SUB), jnp.int32),
        pltpu.VMEM((CHUNK,), jnp.float32),
        pltpu.VMEM((CHUNK,), jnp.float32),
        pltpu.VMEM((16,), jnp.float32),
        pltpu.VMEM((632,), jnp.float32),
        pltpu.VMEM((SUB, D), jnp.bfloat16),
        pltpu.VMEM((SUB, D), jnp.bfloat16),
        pltpu.VMEM((SUB, D), jnp.float32),
        pltpu.VMEM((SUB, D), jnp.float32),
        pltpu.SemaphoreType.DMA,
        pltpu.SemaphoreType.DMA,
        pltpu.SemaphoreType.DMA,
        pltpu.SemaphoreType.DMA,
        pltpu.SemaphoreType.DMA,
        pltpu.SemaphoreType.DMA,
    ],
)(_sc_body)


def _fin_body(u_ref, dx_ref, o_ref):
    u = u_ref[0] + u_ref[1]
    den = dx_ref[0] + dx_ref[1]
    o_ref[...] = u / (den + jnp.float32(1e-16))


def _finalize(u, dx):
    blk = N // 10
    return pl.pallas_call(
        _fin_body,
        grid=(10,),
        in_specs=[
            pl.BlockSpec((NC, blk, D), lambda i: (0, i, 0)),
            pl.BlockSpec((NC, blk, D), lambda i: (0, i, 0)),
        ],
        out_specs=pl.BlockSpec((blk, D), lambda i: (i, 0)),
        out_shape=jax.ShapeDtypeStruct((N, D), jnp.float32),
    )(u, dx)


def kernel(h, edge_embed, W_fc, W_attn, W_feat, edge_index):
    a_src = W_attn[0, 0:D]
    a_dst = W_attn[0, D:2 * D]
    a_feat = W_attn[0, 2 * D:3 * D]
    c = jnp.dot(a_feat, W_feat[:, 0])
    cvec = jnp.full((16,), c, jnp.float32)
    A2 = jnp.stack([a_src, a_dst], axis=1)

    z, s2 = _matmuls(h, W_fc.T, A2)
    s_src = s2[:, 0]
    s_dst = s2[:, 1]
    zb = (z.reshape(N, 4, 2, 16).transpose(0, 1, 3, 2).reshape(N, D)
          .astype(jnp.bfloat16))

    src = edge_index[0]
    dst = edge_index[1]
    emb = edge_embed[:, 0]

    u, dx = _sc_edges(zb, s_src, s_dst, src, dst, emb, cvec)
    return _finalize(u, dx)


# speedup vs baseline: 1.5308x; 1.5308x over previous
import functools

import jax
import jax.numpy as jnp
from jax import lax
from jax.experimental import pallas as pl
from jax.experimental.pallas import tpu as pltpu
from jax.experimental.pallas import tpu_sc as plsc

N = 10000
NDEN = 10240
E = 320000
D = 128
NC = 2
NS = 16
NW = NC * NS
SUB = 80
NSUB = 16
CHUNK = SUB * NSUB
NSTRIPE = E // CHUNK
ITERS = (NSTRIPE + NW - 1) // NW
NGRP = SUB // 16
ROWS_PER_TILE = N // NS
NV = D // 16


def _mm_body(h_ref, wT_ref, a2_ref, z_ref, s2_ref):
    z = lax.dot_general(
        h_ref[...], wT_ref[...], (((1,), (0,)), ((), ())),
        precision=lax.Precision.HIGHEST, preferred_element_type=jnp.float32)
    s2_ref[...] = lax.dot_general(
        z, a2_ref[...], (((1,), (0,)), ((), ())),
        precision=lax.Precision.HIGHEST, preferred_element_type=jnp.float32)
    z_ref[...] = z


def _matmuls(h, W_fcT, A2):
    return pl.pallas_call(
        _mm_body,
        out_shape=[
            jax.ShapeDtypeStruct((N, D), jnp.float32),
            jax.ShapeDtypeStruct((N, 2), jnp.float32),
        ],
    )(h, W_fcT, A2)


def _sc_body(z_hbm, ssrc_hbm, sdst_hbm, src_hbm, dst_hbm, emb_hbm, cvec_hbm,
             u_out, denx_out, u_acc, den_acc, ssrc_v, sdst_v,
             srcq, dstq, embb, wb, cvb, denl, rowsA, rowsB,
             isem, gsemA, gsemB, ssemA, ssemB, dsem):
    cc = lax.axis_index("c")
    tid = lax.axis_index("s")
    wid = cc * NS + tid

    pltpu.sync_copy(ssrc_hbm, ssrc_v)
    pltpu.sync_copy(sdst_hbm, sdst_v)
    pltpu.sync_copy(cvec_hbm, cvb)
    cval = cvb[...]

    def _zrow(r, _):
        for k in range(NV):
            rowsA[r, pl.ds(k * 16, 16)] = jnp.zeros((16,), jnp.float32)
        return _
    lax.fori_loop(0, SUB, _zrow, None)
    r0 = tid * ROWS_PER_TILE
    for q in range(ROWS_PER_TILE // SUB):
        pltpu.sync_copy(rowsA, u_acc.at[pl.ds(r0 + q * SUB, SUB)])
    rem = ROWS_PER_TILE % SUB
    pltpu.sync_copy(rowsA.at[pl.ds(0, rem)],
                    u_acc.at[pl.ds(r0 + (ROWS_PER_TILE // SUB) * SUB, rem)])
    d0 = tid * (NDEN // NS)
    for q in range(NDEN // NS // D):
        pltpu.sync_copy(rowsA.at[0], den_acc.at[pl.ds(d0 + q * D, D)])
    plsc.subcore_barrier()

    rbufs = [rowsA, rowsB]
    gsems = [gsemA, gsemB]
    ssems = [ssemA, ssemB]
    sqs = [srcq.at[q] for q in range(NSUB)]
    dqs = [dstq.at[q] for q in range(NSUB)]

    def _chunk(jj, _):
        j = jj * NW + wid

        @pl.when(j < NSTRIPE)
        def _():
            off = j * CHUNK
            hs = [pltpu.async_copy(src_hbm.at[pl.ds(off + q * SUB, SUB)],
                                   sqs[q], isem) for q in range(NSUB)]
            hs += [pltpu.async_copy(dst_hbm.at[pl.ds(off + q * SUB, SUB)],
                                    dqs[q], isem) for q in range(NSUB)]
            hs.append(pltpu.async_copy(emb_hbm.at[pl.ds(off, CHUNK)], embb,
                                       isem))
            for h in hs:
                h.wait()

            gh = [None] * NSUB
            sh = [None] * NSUB
            dh = [None] * NSUB
            gh[0] = pltpu.async_copy(z_hbm.at[sqs[0]], rbufs[0], gsems[0])
            for q in range(NSUB):
                rb = rbufs[q % 2]
                if q + 1 < NSUB:
                    if q >= 1:
                        sh[q - 1].wait()
                    gh[q + 1] = pltpu.async_copy(
                        z_hbm.at[sqs[q + 1]], rbufs[(q + 1) % 2],
                        gsems[(q + 1) % 2])

                for t in range(NGRP):
                    t16 = t * 16
                    sv = sqs[q][pl.ds(t16, 16)]
                    dv = dqs[q][pl.ds(t16, 16)]
                    s1 = plsc.load_gather(ssrc_v, [sv])
                    s2 = plsc.load_gather(sdst_v, [dv])
                    em = embb[pl.ds(q * SUB + t16, 16)]
                    e = s1 + s2 + cval * em
                    e = jnp.where(e >= 0.0, e, e * jnp.float32(0.01))
                    wb[pl.ds(q * SUB + t16, 16)] = jnp.exp(e)

                dh[q] = pltpu.async_copy(wb.at[pl.ds(q * SUB, SUB)],
                                         den_acc.at[dqs[q]], dsem, add=True)
                gh[q].wait()

                def _scale(r2, _s):
                    r = r2 * 2
                    wr0 = plsc.load_gather(
                        wb, [jnp.zeros((16,), jnp.int32) + (q * SUB + r)])
                    wr1 = plsc.load_gather(
                        wb, [jnp.zeros((16,), jnp.int32) + (q * SUB + r + 1)])
                    for k in range(NV):
                        rb[r, pl.ds(k * 16, 16)] = (
                            rb[r, pl.ds(k * 16, 16)] * wr0)
                    for k in range(NV):
                        rb[r + 1, pl.ds(k * 16, 16)] = (
                            rb[r + 1, pl.ds(k * 16, 16)] * wr1)
                    return _s
                lax.fori_loop(0, SUB // 2, _scale, None)

                sh[q] = pltpu.async_copy(rb, u_acc.at[dqs[q]],
                                         ssems[q % 2], add=True)
            sh[NSUB - 2].wait()
            sh[NSUB - 1].wait()
            for q in range(NSUB):
                dh[q].wait()
        return _

    lax.fori_loop(0, ITERS, _chunk, None)
    plsc.subcore_barrier()

    pltpu.sync_copy(u_acc.at[pl.ds(r0, ROWS_PER_TILE)],
                    u_out.at[cc, pl.ds(r0, ROWS_PER_TILE)])

    a0 = (r0 // 8) * 8
    doff = r0 - a0
    pltpu.sync_copy(den_acc.at[pl.ds(a0, 632)], denl)
    nblk = ROWS_PER_TILE // SUB
    for b in range(nblk + 1):
        cnt = SUB if b < nblk else ROWS_PER_TILE % SUB
        def _exp(r, _, b=b, cnt=cnt):
            dv = plsc.load_gather(
                denl, [jnp.zeros((16,), jnp.int32) + (doff + b * SUB + r)])
            for k in range(NV):
                rowsA[r, pl.ds(k * 16, 16)] = dv
            return _
        lax.fori_loop(0, cnt, _exp, None)
        pltpu.sync_copy(rowsA.at[pl.ds(0, cnt)],
                        denx_out.at[cc, pl.ds(r0 + b * SUB, cnt)])


_sc_edges = functools.partial(
    pl.kernel,
    out_type=[
        jax.ShapeDtypeStruct((NC, N, D), jnp.float32),
        jax.ShapeDtypeStruct((NC, N, D), jnp.float32),
    ],
    mesh=plsc.VectorSubcoreMesh(core_axis_name="c", subcore_axis_name="s"),
    compiler_params=pltpu.CompilerParams(
        use_tc_tiling_on_sc=False, needs_layout_passes=False),
    scratch_types=[
        pltpu.VMEM_SHARED((N, D), jnp.float32),
        pltpu.VMEM_SHARED((NDEN,), jnp.float32),
        pltpu.VMEM((N,), jnp.float32),
        pltpu.VMEM((N,), jnp.float32),
        pltpu.VMEM((NSUB, SUB), jnp.int32),
        pltpu.VMEM((NSUB, SUB), jnp.int32),
        pltpu.VMEM((CHUNK,), jnp.float32),
        pltpu.VMEM((CHUNK,), jnp.float32),
        pltpu.VMEM((16,), jnp.float32),
        pltpu.VMEM((632,), jnp.float32),
        pltpu.VMEM((SUB, D), jnp.float32),
        pltpu.VMEM((SUB, D), jnp.float32),
        pltpu.SemaphoreType.DMA,
        pltpu.SemaphoreType.DMA,
        pltpu.SemaphoreType.DMA,
        pltpu.SemaphoreType.DMA,
        pltpu.SemaphoreType.DMA,
        pltpu.SemaphoreType.DMA,
    ],
)(_sc_body)


def _fin_body(u_ref, dx_ref, o_ref):
    u = u_ref[0] + u_ref[1]
    den = dx_ref[0] + dx_ref[1]
    o_ref[...] = u / (den + jnp.float32(1e-16))


def _finalize(u, dx):
    blk = N // 10
    return pl.pallas_call(
        _fin_body,
        grid=(10,),
        in_specs=[
            pl.BlockSpec((NC, blk, D), lambda i: (0, i, 0)),
            pl.BlockSpec((NC, blk, D), lambda i: (0, i, 0)),
        ],
        out_specs=pl.BlockSpec((blk, D), lambda i: (i, 0)),
        out_shape=jax.ShapeDtypeStruct((N, D), jnp.float32),
    )(u, dx)


def kernel(h, edge_embed, W_fc, W_attn, W_feat, edge_index):
    a_src = W_attn[0, 0:D]
    a_dst = W_attn[0, D:2 * D]
    a_feat = W_attn[0, 2 * D:3 * D]
    c = jnp.dot(a_feat, W_feat[:, 0])
    cvec = jnp.full((16,), c, jnp.float32)
    A2 = jnp.stack([a_src, a_dst], axis=1)

    z, s2 = _matmuls(h, W_fc.T, A2)
    s_src = s2[:, 0]
    s_dst = s2[:, 1]

    src = edge_index[0]
    dst = edge_index[1]
    emb = edge_embed[:, 0]

    u, dx = _sc_edges(z, s_src, s_dst, src, dst, emb, cvec)
    return _finalize(u, dx)
